# baseline (speedup 1.0000x reference)
import jax
import jax.numpy as jnp
from jax.experimental import pallas as pl
from jax.experimental.pallas import tpu as pltpu

_BM = 200


def _spmm_kernel(adj_ref, x_ref, o_ref):
    o_ref[...] = jnp.dot(adj_ref[...], x_ref[...],
                         preferred_element_type=jnp.float32)


def kernel(x, adj):
    n, d = x.shape
    bm = _BM if n % _BM == 0 else n
    return pl.pallas_call(
        _spmm_kernel,
        grid=(n // bm,),
        in_specs=[
            pl.BlockSpec((bm, n), lambda i: (i, 0)),
            pl.BlockSpec((n, d), lambda i: (0, 0)),
        ],
        out_specs=pl.BlockSpec((bm, d), lambda i: (i, 0)),
        out_shape=jax.ShapeDtypeStruct((n, d), jnp.float32),
        compiler_params=pltpu.CompilerParams(
            dimension_semantics=("parallel",),
        ),
    )(adj, x)


# speedup vs baseline: 1.0029x; 1.0029x over previous
import jax
import jax.numpy as jnp
from jax.experimental import pallas as pl

_BM = 200


def _spmm_kernel(adj_ref, x_ref, o_ref):
    o_ref[...] = jnp.dot(adj_ref[...], x_ref[...],
                         preferred_element_type=jnp.float32)


def kernel(x, adj):
    n, d = x.shape
    bm = _BM if n % _BM == 0 else n
    return pl.pallas_call(
        _spmm_kernel,
        grid=(n // bm,),
        in_specs=[
            pl.BlockSpec((bm, n), lambda i: (i, 0)),
            pl.BlockSpec((n, d), lambda i: (0, 0)),
        ],
        out_specs=pl.BlockSpec((bm, d), lambda i: (i, 0)),
        out_shape=jax.ShapeDtypeStruct((n, d), jnp.float32),
    )(adj, x)
